# baseline (speedup 1.0000x reference)
import functools

import jax
import jax.numpy as jnp
import numpy as np
from jax import lax
from jax.experimental import pallas as pl
from jax.experimental.pallas import tpu as pltpu
from jax.experimental.pallas import tpu_sc as plsc

TOPK = 8
WIN = 25
T = 512
NFFT = 1024
K = NFFT // 2 + 1
TC_BLOCK = 256
D = 64


def _constants():
    pad = WIN // 2
    M = np.zeros((T, T), dtype=np.float64)
    for tau in range(T):
        for j in range(-pad, pad + 1):
            src = min(max(tau + j, 0), T - 1)
            M[src, tau] += 1.0 / WIN
    A = np.eye(T, dtype=np.float64) - M

    t = np.arange(T, dtype=np.int64)[:, None]
    k = np.arange(K, dtype=np.int64)[None, :]
    ang = 2.0 * np.pi * ((t * k) % NFFT).astype(np.float64) / NFFT
    C = np.cos(ang)
    S = np.sin(ang)

    kk = np.arange(K, dtype=np.int64)[:, None]
    tt = np.arange(T, dtype=np.int64)[None, :]
    ang2 = 2.0 * np.pi * ((kk * tt) % NFFT).astype(np.float64) / NFFT
    w = np.full((K, 1), 2.0 / NFFT, dtype=np.float64)
    w[0, 0] = 1.0 / NFFT
    w[K - 1, 0] = 1.0 / NFFT
    Ci = np.cos(ang2) * w

    f32 = lambda a: jnp.asarray(a, dtype=jnp.float32)
    return f32(A), f32(C), f32(S), f32(Ci)


def _tc_body(x_ref, a_ref, c_ref, s_ref, ci_ref, out_ref):
    dot = functools.partial(
        jax.lax.dot,
        precision=jax.lax.Precision.HIGHEST,
        preferred_element_type=jnp.float32,
    )
    x = x_ref[...]
    d = dot(x, a_ref[...])
    re = dot(d, c_ref[...])
    im = dot(d, s_ref[...])
    p = re * re + im * im
    ac = dot(p, ci_ref[...])

    lane = jax.lax.broadcasted_iota(jnp.int32, ac.shape, 1)
    ac = jnp.where(lane == 0, jnp.float32(-1e9), ac)

    b = ac.shape[0]
    out_lane = jax.lax.broadcasted_iota(jnp.int32, (b, TOPK), 1)
    out = jnp.zeros((b, TOPK), dtype=jnp.int32)
    work = ac
    for kth in range(TOPK):
        m = jnp.max(work, axis=1, keepdims=True)
        arg = jnp.min(
            jnp.where(work == m, lane, jnp.int32(T)), axis=1, keepdims=True
        )
        out = jnp.where(out_lane == kth, jnp.broadcast_to(arg, (b, TOPK)), out)
        work = jnp.where(lane == arg, jnp.float32(-3e38), work)
    out_ref[...] = out


def _tc_backend(xbar, A, C, S, Ci):
    BN = xbar.shape[0]
    grid = (BN // TC_BLOCK,)
    return pl.pallas_call(
        _tc_body,
        grid=grid,
        in_specs=[
            pl.BlockSpec((TC_BLOCK, T), lambda i: (i, 0)),
            pl.BlockSpec((T, T), lambda i: (0, 0)),
            pl.BlockSpec((T, K), lambda i: (0, 0)),
            pl.BlockSpec((T, K), lambda i: (0, 0)),
            pl.BlockSpec((K, T), lambda i: (0, 0)),
        ],
        out_specs=pl.BlockSpec((TC_BLOCK, TOPK), lambda i: (i, 0)),
        out_shape=jax.ShapeDtypeStruct((BN, TOPK), jnp.int32),
    )(xbar, A, C, S, Ci)


def _sc_mean(X):
    BN = X.shape[0]
    info = plsc.get_sparse_core_info()
    nw = info.num_cores * info.num_subcores
    rows_per_w = BN // nw
    mesh = plsc.VectorSubcoreMesh(core_axis_name="c", subcore_axis_name="s")

    @functools.partial(
        pl.kernel,
        out_type=jax.ShapeDtypeStruct((BN, T), jnp.float32),
        mesh=mesh,
        scratch_types=[
            pltpu.VMEM((T // 4, D), jnp.float32),
            pltpu.VMEM((T // 4, D), jnp.float32),
            pltpu.VMEM((T // 4, D), jnp.float32),
            pltpu.VMEM((T // 4, D), jnp.float32),
            pltpu.VMEM((8, T), jnp.float32),
            pltpu.SemaphoreType.DMA,
            pltpu.SemaphoreType.DMA,
            pltpu.SemaphoreType.DMA,
            pltpu.SemaphoreType.DMA,
        ],
        compiler_params=pltpu.CompilerParams(
            use_tc_tiling_on_sc=True, needs_layout_passes=False
        ),
    )
    def sc_kernel(x_hbm, out_hbm, b0, b1, b2, b3, obuf, s0, s1, s2, s3):
        wid = lax.axis_index("s") * info.num_cores + lax.axis_index("c")
        base = wid * rows_per_w
        bufs = (b0, b1, b2, b3)
        sems = (s0, s1, s2, s3)
        lane16 = lax.iota(jnp.int32, 16)

        Q = T // 4
        NACC = 8

        def reduce_quarter(buf, j, q):
            def group(g, _):
                tvec = lane16 + g * 16
                accs = [jnp.zeros((16,), jnp.float32) for _ in range(NACC)]
                for dd in range(D):
                    dvec = jnp.full((16,), dd, jnp.int32)
                    accs[dd % NACC] = accs[dd % NACC] + plsc.load_gather(
                        buf, [tvec, dvec]
                    )
                while len(accs) > 1:
                    accs = [
                        accs[z] + accs[z + 1] for z in range(0, len(accs), 2)
                    ]
                obuf[j, pl.ds(q * Q + g * 16, 16)] = accs[0] * jnp.float32(
                    1.0 / D
                )
                return 0

            lax.fori_loop(0, Q // 16, group, 0)

        for q in range(4):
            pltpu.async_copy(x_hbm.at[base, pl.ds(q * Q, Q)], bufs[q], sems[q])

        def row_step(i, _):
            r = base + i
            j = lax.rem(i, 8)
            for q in range(4):
                pltpu.make_async_copy(
                    x_hbm.at[r, pl.ds(q * Q, Q)], bufs[q], sems[q]
                ).wait()
                reduce_quarter(bufs[q], j, q)

                @pl.when(i + 1 < rows_per_w)
                def _():
                    pltpu.async_copy(
                        x_hbm.at[r + 1, pl.ds(q * Q, Q)], bufs[q], sems[q]
                    )

            @pl.when(j == 7)
            def _():
                start = pl.multiple_of(r - 7, 8)
                pltpu.sync_copy(obuf, out_hbm.at[pl.ds(start, 8)])

            return 0

        lax.fori_loop(0, rows_per_w, row_step, 0)

    return sc_kernel(X)


def kernel(X):
    A, C, S, Ci = _constants()
    xbar = _sc_mean(X)
    return _tc_backend(xbar, A, C, S, Ci)


# speedup vs baseline: 1.8873x; 1.8873x over previous
import functools

import jax
import jax.numpy as jnp
import numpy as np
from jax import lax
from jax.experimental import pallas as pl
from jax.experimental.pallas import tpu as pltpu
from jax.experimental.pallas import tpu_sc as plsc

TOPK = 8
WIN = 25
T = 512
NFFT = 1024
K = NFFT // 2 + 1
TC_BLOCK = 256
D = 64


def _constants():
    pad = WIN // 2
    M = np.zeros((T, T), dtype=np.float64)
    for tau in range(T):
        for j in range(-pad, pad + 1):
            src = min(max(tau + j, 0), T - 1)
            M[src, tau] += 1.0 / WIN
    A = np.eye(T, dtype=np.float64) - M

    t = np.arange(T, dtype=np.int64)[:, None]
    k = np.arange(K, dtype=np.int64)[None, :]
    ang = 2.0 * np.pi * ((t * k) % NFFT).astype(np.float64) / NFFT
    C = np.cos(ang)
    S = np.sin(ang)

    kk = np.arange(K, dtype=np.int64)[:, None]
    tt = np.arange(T, dtype=np.int64)[None, :]
    ang2 = 2.0 * np.pi * ((kk * tt) % NFFT).astype(np.float64) / NFFT
    w = np.full((K, 1), 2.0 / NFFT, dtype=np.float64)
    w[0, 0] = 1.0 / NFFT
    w[K - 1, 0] = 1.0 / NFFT
    Ci = np.cos(ang2) * w

    f32 = lambda a: jnp.asarray(a, dtype=jnp.float32)
    return f32(A), f32(C), f32(S), f32(Ci)


def _tc_body(x_ref, a_ref, c_ref, s_ref, ci_ref, out_ref):
    dot = functools.partial(
        jax.lax.dot,
        precision=jax.lax.Precision.HIGHEST,
        preferred_element_type=jnp.float32,
    )
    x = x_ref[...]
    d = dot(x, a_ref[...])
    re = dot(d, c_ref[...])
    im = dot(d, s_ref[...])
    p = re * re + im * im
    ac = dot(p, ci_ref[...])

    lane = jax.lax.broadcasted_iota(jnp.int32, ac.shape, 1)
    ac = jnp.where(lane == 0, jnp.float32(-1e9), ac)

    b = ac.shape[0]
    out_lane = jax.lax.broadcasted_iota(jnp.int32, (b, TOPK), 1)
    out = jnp.zeros((b, TOPK), dtype=jnp.int32)
    work = ac
    for kth in range(TOPK):
        m = jnp.max(work, axis=1, keepdims=True)
        arg = jnp.min(
            jnp.where(work == m, lane, jnp.int32(T)), axis=1, keepdims=True
        )
        out = jnp.where(out_lane == kth, jnp.broadcast_to(arg, (b, TOPK)), out)
        work = jnp.where(lane == arg, jnp.float32(-3e38), work)
    out_ref[...] = out


def _tc_backend(xbar, A, C, S, Ci):
    BN = xbar.shape[0]
    grid = (BN // TC_BLOCK,)
    return pl.pallas_call(
        _tc_body,
        grid=grid,
        in_specs=[
            pl.BlockSpec((TC_BLOCK, T), lambda i: (i, 0)),
            pl.BlockSpec((T, T), lambda i: (0, 0)),
            pl.BlockSpec((T, K), lambda i: (0, 0)),
            pl.BlockSpec((T, K), lambda i: (0, 0)),
            pl.BlockSpec((K, T), lambda i: (0, 0)),
        ],
        out_specs=pl.BlockSpec((TC_BLOCK, TOPK), lambda i: (i, 0)),
        out_shape=jax.ShapeDtypeStruct((BN, TOPK), jnp.int32),
    )(xbar, A, C, S, Ci)


def _sc_mean(X):
    BN = X.shape[0]
    info = plsc.get_sparse_core_info()
    nw = info.num_cores * info.num_subcores
    rows_per_w = BN // nw
    mesh = plsc.VectorSubcoreMesh(core_axis_name="c", subcore_axis_name="s")

    @functools.partial(
        pl.kernel,
        out_type=jax.ShapeDtypeStruct((BN, T), jnp.float32),
        mesh=mesh,
        scratch_types=[
            pltpu.VMEM((T // 4, D), jnp.float32),
            pltpu.VMEM((T // 4, D), jnp.float32),
            pltpu.VMEM((T // 4, D), jnp.float32),
            pltpu.VMEM((T // 4, D), jnp.float32),
            pltpu.VMEM((8, T), jnp.float32),
            pltpu.SemaphoreType.DMA,
            pltpu.SemaphoreType.DMA,
            pltpu.SemaphoreType.DMA,
            pltpu.SemaphoreType.DMA,
        ],
        compiler_params=pltpu.CompilerParams(
            use_tc_tiling_on_sc=True, needs_layout_passes=False
        ),
    )
    def sc_kernel(x_hbm, out_hbm, b0, b1, b2, b3, obuf, s0, s1, s2, s3):
        wid = lax.axis_index("s") * info.num_cores + lax.axis_index("c")
        base = wid * rows_per_w
        bufs = (b0, b1, b2, b3)
        sems = (s0, s1, s2, s3)
        lane16 = lax.iota(jnp.int32, 16)

        Q = T // 4
        NACC = 8

        def reduce_quarter(buf, j, q):
            def group(g, _):
                tvec = lane16 + g * 16
                accs = [jnp.zeros((16,), jnp.float32) for _ in range(NACC)]
                for dd in range(D):
                    dvec = jnp.bitwise_and(lane16 + dd, D - 1)
                    accs[dd % NACC] = accs[dd % NACC] + plsc.load_gather(
                        buf, [tvec, dvec]
                    )
                while len(accs) > 1:
                    accs = [
                        accs[z] + accs[z + 1] for z in range(0, len(accs), 2)
                    ]
                obuf[j, pl.ds(q * Q + g * 16, 16)] = accs[0] * jnp.float32(
                    1.0 / D
                )
                return 0

            lax.fori_loop(0, Q // 16, group, 0)

        for q in range(4):
            pltpu.async_copy(x_hbm.at[base, pl.ds(q * Q, Q)], bufs[q], sems[q])

        def row_step(i, _):
            r = base + i
            j = lax.rem(i, 8)
            for q in range(4):
                pltpu.make_async_copy(
                    x_hbm.at[r, pl.ds(q * Q, Q)], bufs[q], sems[q]
                ).wait()
                reduce_quarter(bufs[q], j, q)

                @pl.when(i + 1 < rows_per_w)
                def _():
                    pltpu.async_copy(
                        x_hbm.at[r + 1, pl.ds(q * Q, Q)], bufs[q], sems[q]
                    )

            @pl.when(j == 7)
            def _():
                start = pl.multiple_of(r - 7, 8)
                pltpu.sync_copy(obuf, out_hbm.at[pl.ds(start, 8)])

            return 0

        lax.fori_loop(0, rows_per_w, row_step, 0)

    return sc_kernel(X)


def kernel(X):
    A, C, S, Ci = _constants()
    xbar = _sc_mean(X)
    return _tc_backend(xbar, A, C, S, Ci)


# speedup vs baseline: 10.9244x; 5.7884x over previous
import functools

import jax
import jax.numpy as jnp
import numpy as np
from jax.experimental import pallas as pl

TOPK = 8
WIN = 25
T = 512
NFFT = 1024
K = NFFT // 2 + 1
BLOCK = 128
D = 64


def _constants():
    pad = WIN // 2
    M = np.zeros((T, T), dtype=np.float64)
    for tau in range(T):
        for j in range(-pad, pad + 1):
            src = min(max(tau + j, 0), T - 1)
            M[src, tau] += 1.0 / WIN
    A = np.eye(T, dtype=np.float64) - M

    t = np.arange(T, dtype=np.int64)[:, None]
    k = np.arange(K, dtype=np.int64)[None, :]
    ang = 2.0 * np.pi * ((t * k) % NFFT).astype(np.float64) / NFFT
    C = np.cos(ang)
    S = np.sin(ang)

    kk = np.arange(K, dtype=np.int64)[:, None]
    tt = np.arange(T, dtype=np.int64)[None, :]
    ang2 = 2.0 * np.pi * ((kk * tt) % NFFT).astype(np.float64) / NFFT
    w = np.full((K, 1), 2.0 / NFFT, dtype=np.float64)
    w[0, 0] = 1.0 / NFFT
    w[K - 1, 0] = 1.0 / NFFT
    Ci = np.cos(ang2) * w

    f32 = lambda a: jnp.asarray(a, dtype=jnp.float32)
    return f32(A), f32(C), f32(S), f32(Ci)


def _body(x_ref, a_ref, c_ref, s_ref, ci_ref, out_ref):
    dot = functools.partial(
        jax.lax.dot,
        precision=jax.lax.Precision.HIGHEST,
        preferred_element_type=jnp.float32,
    )
    x = jnp.mean(x_ref[...], axis=1)
    d = dot(x, a_ref[...])
    re = dot(d, c_ref[...])
    im = dot(d, s_ref[...])
    p = re * re + im * im
    ac = dot(p, ci_ref[...])

    lane = jax.lax.broadcasted_iota(jnp.int32, ac.shape, 1)
    ac = jnp.where(lane == 0, jnp.float32(-1e9), ac)

    b = ac.shape[0]
    out_lane = jax.lax.broadcasted_iota(jnp.int32, (b, TOPK), 1)
    out = jnp.zeros((b, TOPK), dtype=jnp.int32)
    work = ac
    for kth in range(TOPK):
        m = jnp.max(work, axis=1, keepdims=True)
        arg = jnp.min(
            jnp.where(work == m, lane, jnp.int32(T)), axis=1, keepdims=True
        )
        out = jnp.where(out_lane == kth, jnp.broadcast_to(arg, (b, TOPK)), out)
        work = jnp.where(lane == arg, jnp.float32(-3e38), work)
    out_ref[...] = out


def kernel(X):
    BN, t, d = X.shape
    A, C, S, Ci = _constants()
    Xt = jnp.transpose(X, (0, 2, 1))
    grid = (BN // BLOCK,)
    return pl.pallas_call(
        _body,
        grid=grid,
        in_specs=[
            pl.BlockSpec((BLOCK, d, t), lambda i: (i, 0, 0)),
            pl.BlockSpec((T, T), lambda i: (0, 0)),
            pl.BlockSpec((T, K), lambda i: (0, 0)),
            pl.BlockSpec((T, K), lambda i: (0, 0)),
            pl.BlockSpec((K, T), lambda i: (0, 0)),
        ],
        out_specs=pl.BlockSpec((BLOCK, TOPK), lambda i: (i, 0)),
        out_shape=jax.ShapeDtypeStruct((BN, TOPK), jnp.int32),
    )(Xt, A, C, S, Ci)
